# baseline (speedup 1.0000x reference)
import functools
import math

import jax
import jax.numpy as jnp
from jax.experimental import pallas as pl

_N = 10000
_E = 320000
_H = 128
_NEG = 0.2
_RATIO = 0.8

_NODE_BLK = 1000
_EDGE_BLK = 2048
_EPAD = 331776
_PACK_ROWS = _EPAD // 128
_PACK_BLK = 648


def _linear_k(x_ref, w_ref, b_ref, o_ref, *, act):
    y = jnp.dot(x_ref[...], w_ref[...], preferred_element_type=jnp.float32)
    y = y + b_ref[...]
    if act == "relu":
        y = jnp.maximum(y, 0.0)
    o_ref[...] = y


def _linear(x, W, b, act=None, blk=_NODE_BLK):
    M, K = x.shape
    O = W.shape[1]
    return pl.pallas_call(
        functools.partial(_linear_k, act=act),
        grid=(M // blk,),
        in_specs=[
            pl.BlockSpec((blk, K), lambda i: (i, 0)),
            pl.BlockSpec((K, O), lambda i: (0, 0)),
            pl.BlockSpec((1, O), lambda i: (0, 0)),
        ],
        out_specs=pl.BlockSpec((blk, O), lambda i: (i, 0)),
        out_shape=jax.ShapeDtypeStruct((M, O), jnp.float32),
    )(x, W, b.reshape(1, O))


def _bias_act_k(x_ref, b_ref, o_ref, *, act):
    y = x_ref[...] + b_ref[...]
    if act == "relu":
        y = jnp.maximum(y, 0.0)
    o_ref[...] = y


def _bias_act(x, b, act=None, blk=_NODE_BLK):
    M, O = x.shape
    return pl.pallas_call(
        functools.partial(_bias_act_k, act=act),
        grid=(M // blk,),
        in_specs=[
            pl.BlockSpec((blk, O), lambda i: (i, 0)),
            pl.BlockSpec((1, O), lambda i: (0, 0)),
        ],
        out_specs=pl.BlockSpec((blk, O), lambda i: (i, 0)),
        out_shape=jax.ShapeDtypeStruct((M, O), jnp.float32),
    )(x, b.reshape(1, O))


def _mul3_k(a_ref, b_ref, c_ref, o_ref):
    o_ref[...] = a_ref[...] * b_ref[...] * c_ref[...]


def _lemsg_k(a_ref, b_ref, c_ref, o_ref):
    o_ref[...] = (a_ref[...] - b_ref[...]) * c_ref[...]


def _expsub_k(a_ref, b_ref, o_ref):
    o_ref[...] = jnp.exp(a_ref[...] - b_ref[...])


def _packed2(kern, a, b):
    return pl.pallas_call(
        kern,
        grid=(_PACK_ROWS // _PACK_BLK,),
        in_specs=[pl.BlockSpec((_PACK_BLK, 128), lambda i: (i, 0))] * 2,
        out_specs=pl.BlockSpec((_PACK_BLK, 128), lambda i: (i, 0)),
        out_shape=jax.ShapeDtypeStruct((_PACK_ROWS, 128), jnp.float32),
    )(a, b)


def _packed3(kern, a, b, c):
    return pl.pallas_call(
        kern,
        grid=(_PACK_ROWS // _PACK_BLK,),
        in_specs=[pl.BlockSpec((_PACK_BLK, 128), lambda i: (i, 0))] * 3,
        out_specs=pl.BlockSpec((_PACK_BLK, 128), lambda i: (i, 0)),
        out_shape=jax.ShapeDtypeStruct((_PACK_ROWS, 128), jnp.float32),
    )(a, b, c)


def _rowscale_k(x_ref, s_ref, o_ref):
    o_ref[...] = x_ref[...] * s_ref[...]


def _rowscale(xf, s_col):
    return pl.pallas_call(
        _rowscale_k,
        grid=(_EPAD // _EDGE_BLK,),
        in_specs=[
            pl.BlockSpec((_EDGE_BLK, 128), lambda i: (i, 0)),
            pl.BlockSpec((_EDGE_BLK, 1), lambda i: (i, 0)),
        ],
        out_specs=pl.BlockSpec((_EDGE_BLK, 128), lambda i: (i, 0)),
        out_shape=jax.ShapeDtypeStruct((_EPAD, 128), jnp.float32),
    )(xf, s_col)


def _attmsg_k(x_ref, e_ref, d_ref, o_ref):
    o_ref[...] = x_ref[...] * (e_ref[...] / (d_ref[...] + 1e-16))


def _attmsg(xf, e_col, d_col):
    return pl.pallas_call(
        _attmsg_k,
        grid=(_EPAD // _EDGE_BLK,),
        in_specs=[
            pl.BlockSpec((_EDGE_BLK, 128), lambda i: (i, 0)),
            pl.BlockSpec((_EDGE_BLK, 1), lambda i: (i, 0)),
            pl.BlockSpec((_EDGE_BLK, 1), lambda i: (i, 0)),
        ],
        out_specs=pl.BlockSpec((_EDGE_BLK, 128), lambda i: (i, 0)),
        out_shape=jax.ShapeDtypeStruct((_EPAD, 128), jnp.float32),
    )(xf, e_col, d_col)


def _score_k(mq_ref, xpj_ref, wa1_ref, wa2_ref, b_ref, o_ref):
    s = jnp.dot(mq_ref[...], wa1_ref[...], preferred_element_type=jnp.float32)
    s = s + jnp.dot(xpj_ref[...], wa2_ref[...], preferred_element_type=jnp.float32)
    s = s + b_ref[0, 0]
    o_ref[...] = jnp.where(s >= 0.0, s, _NEG * s)


def _score(mq_row, xpj, Wa1, Wa2, b_att):
    return pl.pallas_call(
        _score_k,
        grid=(_EPAD // _EDGE_BLK,),
        in_specs=[
            pl.BlockSpec((_EDGE_BLK, 128), lambda i: (i, 0)),
            pl.BlockSpec((_EDGE_BLK, 128), lambda i: (i, 0)),
            pl.BlockSpec((128, 1), lambda i: (0, 0)),
            pl.BlockSpec((128, 1), lambda i: (0, 0)),
            pl.BlockSpec((1, 1), lambda i: (0, 0)),
        ],
        out_specs=pl.BlockSpec((_EDGE_BLK, 1), lambda i: (i, 0)),
        out_shape=jax.ShapeDtypeStruct((_EPAD, 1), jnp.float32),
    )(mq_row, xpj, Wa1, Wa2, b_att.reshape(1, 1))


def kernel(x, edge_index, batch, W_conv1, b_conv1, W_intra, b_intra, W_q, b_q,
           W_att, b_att, Ws1, bs1, Ws2, bs2, Ws3, bs3, Wp1, bp1, Wp2, bp2):
    row0, col0 = edge_index[0], edge_index[1]
    keep = row0 != col0
    drop = jnp.int32(_N)
    row0 = jnp.where(keep, row0, drop)
    col0 = jnp.where(keep, col0, drop)
    loops = jnp.arange(_N, dtype=jnp.int32)
    npad = _EPAD - _E - _N
    padi = jnp.full((npad,), _N, jnp.int32)
    row = jnp.concatenate([row0, loops, padi])
    col = jnp.concatenate([col0, loops, padi])
    ew = jnp.concatenate([jnp.ones((_E + _N,), jnp.float32),
                          jnp.zeros((npad,), jnp.float32)])

    deg = jax.ops.segment_sum(ew, col, num_segments=_N)
    dis = jnp.where(deg > 0, 1.0 / jnp.sqrt(jnp.maximum(deg, 1e-12)), 0.0)
    p = lambda v: v.reshape(_PACK_ROWS, 128)
    norm2 = _packed3(_mul3_k, p(dis[row]), p(ew), p(dis[col]))
    norm_col = norm2.reshape(_EPAD, 1)

    h0 = _linear(x, W_conv1, jnp.zeros_like(b_conv1))
    h = _bias_act(jax.ops.segment_sum(_rowscale(h0[row], norm_col), col,
                                      num_segments=_N), b_conv1, act="relu")

    l2 = _linear(h, W_intra, jnp.zeros_like(b_intra))
    x_pool = _bias_act(jax.ops.segment_sum(_rowscale(l2[row], norm_col), col,
                                           num_segments=_N), b_intra)

    x_pool_j = x_pool[col]
    x_j = h[col]

    X_q = jax.ops.segment_max(x_pool_j, row, num_segments=_N)
    Mq = _linear(X_q, W_q, b_q)
    s_col = _score(Mq[row], x_pool_j, W_att[:_H], W_att[_H:], b_att)
    s = s_col.reshape(_EPAD)

    m = jax.ops.segment_max(s, row, num_segments=_N)
    e2 = _packed2(_expsub_k, p(s), p(m[row]))
    e = e2.reshape(_EPAD)
    d = jax.ops.segment_sum(e, row, num_segments=_N)
    msg = _attmsg(x_j, e.reshape(_EPAD, 1), d[row].reshape(_EPAD, 1))
    out = jax.ops.segment_sum(msg, row, num_segments=_N)

    Wabc = jnp.zeros((_H, 128), jnp.float32)
    Wabc = Wabc.at[:, 0].set(Ws1[:, 0]).at[:, 1].set(Ws2[:, 0]).at[:, 2].set(Ws3[:, 0])
    babc = jnp.zeros((128,), jnp.float32)
    babc = babc.at[0].set(bs1[0]).at[1].set(bs2[0]).at[2].set(bs3[0])
    abc = _linear(out, Wabc, babc)
    a_v, b_v, c_v = abc[:, 0], abc[:, 1], abc[:, 2]
    lem2 = _packed3(_lemsg_k, p(a_v[row]), p(b_v[col]), p(ew))
    agg = jax.ops.segment_sum(lem2.reshape(_EPAD), row, num_segments=_N)
    fitness = jax.nn.sigmoid(agg + c_v)

    k = int(math.ceil(_RATIO * _N))
    _, perm = jax.lax.top_k(fitness, k)
    xp = out[perm] * fitness[perm][:, None]
    batch_p = batch[perm]
    nb = 1
    cnt = jax.ops.segment_sum(jnp.ones((k,), jnp.float32), batch_p, num_segments=nb)
    x_mean = jax.ops.segment_sum(xp, batch_p, num_segments=nb) / jnp.maximum(cnt, 1.0)[:, None]
    x_max = jax.ops.segment_max(xp, batch_p, num_segments=nb)
    xs = jnp.concatenate([x_mean, x_max], axis=-1)
    logits = jax.nn.relu(xs @ Wp1 + bp1) @ Wp2 + bp2
    pred = jax.nn.log_softmax(logits, axis=-1)
    return pred[..., None]


# speedup vs baseline: 1.0540x; 1.0540x over previous
import functools
import math

import jax
import jax.numpy as jnp
from jax.experimental import pallas as pl

_N = 10000
_E = 320000
_H = 128
_NEG = 0.2
_RATIO = 0.8

_NODE_BLK = 1000
_EDGE_BLK = 2048
_EPAD = 331776
_PACK_ROWS = _EPAD // 128
_PACK_BLK = 648


def _linear_k(x_ref, w_ref, b_ref, o_ref, *, act):
    y = jnp.dot(x_ref[...], w_ref[...], preferred_element_type=jnp.float32)
    y = y + b_ref[...]
    if act == "relu":
        y = jnp.maximum(y, 0.0)
    o_ref[...] = y


def _linear(x, W, b, act=None, blk=_NODE_BLK):
    M, K = x.shape
    O = W.shape[1]
    return pl.pallas_call(
        functools.partial(_linear_k, act=act),
        grid=(M // blk,),
        in_specs=[
            pl.BlockSpec((blk, K), lambda i: (i, 0)),
            pl.BlockSpec((K, O), lambda i: (0, 0)),
            pl.BlockSpec((1, O), lambda i: (0, 0)),
        ],
        out_specs=pl.BlockSpec((blk, O), lambda i: (i, 0)),
        out_shape=jax.ShapeDtypeStruct((M, O), jnp.float32),
    )(x, W, b.reshape(1, O))


def _bias_act_k(x_ref, b_ref, o_ref, *, act):
    y = x_ref[...] + b_ref[...]
    if act == "relu":
        y = jnp.maximum(y, 0.0)
    o_ref[...] = y


def _bias_act(x, b, act=None, blk=_NODE_BLK):
    M, O = x.shape
    return pl.pallas_call(
        functools.partial(_bias_act_k, act=act),
        grid=(M // blk,),
        in_specs=[
            pl.BlockSpec((blk, O), lambda i: (i, 0)),
            pl.BlockSpec((1, O), lambda i: (0, 0)),
        ],
        out_specs=pl.BlockSpec((blk, O), lambda i: (i, 0)),
        out_shape=jax.ShapeDtypeStruct((M, O), jnp.float32),
    )(x, b.reshape(1, O))


def _mul3_k(a_ref, b_ref, c_ref, o_ref):
    o_ref[...] = a_ref[...] * b_ref[...] * c_ref[...]


def _lemsg_k(a_ref, b_ref, c_ref, o_ref):
    o_ref[...] = (a_ref[...] - b_ref[...]) * c_ref[...]


def _expsub_k(a_ref, b_ref, o_ref):
    o_ref[...] = jnp.exp(a_ref[...] - b_ref[...])


def _packed2(kern, a, b):
    return pl.pallas_call(
        kern,
        grid=(_PACK_ROWS // _PACK_BLK,),
        in_specs=[pl.BlockSpec((_PACK_BLK, 128), lambda i: (i, 0))] * 2,
        out_specs=pl.BlockSpec((_PACK_BLK, 128), lambda i: (i, 0)),
        out_shape=jax.ShapeDtypeStruct((_PACK_ROWS, 128), jnp.float32),
    )(a, b)


def _packed3(kern, a, b, c):
    return pl.pallas_call(
        kern,
        grid=(_PACK_ROWS // _PACK_BLK,),
        in_specs=[pl.BlockSpec((_PACK_BLK, 128), lambda i: (i, 0))] * 3,
        out_specs=pl.BlockSpec((_PACK_BLK, 128), lambda i: (i, 0)),
        out_shape=jax.ShapeDtypeStruct((_PACK_ROWS, 128), jnp.float32),
    )(a, b, c)


def _rowscale_k(x_ref, s_ref, o_ref):
    o_ref[...] = x_ref[...] * s_ref[...]


def _rowscale(xf, s_col, blk=_NODE_BLK):
    M = xf.shape[0]
    return pl.pallas_call(
        _rowscale_k,
        grid=(M // blk,),
        in_specs=[
            pl.BlockSpec((blk, 128), lambda i: (i, 0)),
            pl.BlockSpec((blk, 1), lambda i: (i, 0)),
        ],
        out_specs=pl.BlockSpec((blk, 128), lambda i: (i, 0)),
        out_shape=jax.ShapeDtypeStruct((M, 128), jnp.float32),
    )(xf, s_col)


def _scale_bias_k(x_ref, s_ref, b_ref, o_ref, *, act):
    y = x_ref[...] * s_ref[...] + b_ref[...]
    if act == "relu":
        y = jnp.maximum(y, 0.0)
    o_ref[...] = y


def _scale_bias(x, s_col, b, act=None, blk=_NODE_BLK):
    M, O = x.shape
    return pl.pallas_call(
        functools.partial(_scale_bias_k, act=act),
        grid=(M // blk,),
        in_specs=[
            pl.BlockSpec((blk, O), lambda i: (i, 0)),
            pl.BlockSpec((blk, 1), lambda i: (i, 0)),
            pl.BlockSpec((1, O), lambda i: (0, 0)),
        ],
        out_specs=pl.BlockSpec((blk, O), lambda i: (i, 0)),
        out_shape=jax.ShapeDtypeStruct((M, O), jnp.float32),
    )(x, s_col, b.reshape(1, O))


def _leaky_add_k(a_ref, b_ref, c_ref, o_ref):
    s = a_ref[...] + b_ref[...] + c_ref[0, 0]
    o_ref[...] = jnp.where(s >= 0.0, s, _NEG * s)


def _attmsg_k(x_ref, e_ref, d_ref, o_ref):
    o_ref[...] = x_ref[...] * (e_ref[...] / (d_ref[...] + 1e-16))


def _attmsg(xf, e_col, d_col):
    return pl.pallas_call(
        _attmsg_k,
        grid=(_EPAD // _EDGE_BLK,),
        in_specs=[
            pl.BlockSpec((_EDGE_BLK, 128), lambda i: (i, 0)),
            pl.BlockSpec((_EDGE_BLK, 1), lambda i: (i, 0)),
            pl.BlockSpec((_EDGE_BLK, 1), lambda i: (i, 0)),
        ],
        out_specs=pl.BlockSpec((_EDGE_BLK, 128), lambda i: (i, 0)),
        out_shape=jax.ShapeDtypeStruct((_EPAD, 128), jnp.float32),
    )(xf, e_col, d_col)


def _score_k(mq_ref, xpj_ref, wa1_ref, wa2_ref, b_ref, o_ref):
    s = jnp.dot(mq_ref[...], wa1_ref[...], preferred_element_type=jnp.float32)
    s = s + jnp.dot(xpj_ref[...], wa2_ref[...], preferred_element_type=jnp.float32)
    s = s + b_ref[0, 0]
    o_ref[...] = jnp.where(s >= 0.0, s, _NEG * s)


def _score(mq_row, xpj, Wa1, Wa2, b_att):
    return pl.pallas_call(
        _score_k,
        grid=(_EPAD // _EDGE_BLK,),
        in_specs=[
            pl.BlockSpec((_EDGE_BLK, 128), lambda i: (i, 0)),
            pl.BlockSpec((_EDGE_BLK, 128), lambda i: (i, 0)),
            pl.BlockSpec((128, 1), lambda i: (0, 0)),
            pl.BlockSpec((128, 1), lambda i: (0, 0)),
            pl.BlockSpec((1, 1), lambda i: (0, 0)),
        ],
        out_specs=pl.BlockSpec((_EDGE_BLK, 1), lambda i: (i, 0)),
        out_shape=jax.ShapeDtypeStruct((_EPAD, 1), jnp.float32),
    )(mq_row, xpj, Wa1, Wa2, b_att.reshape(1, 1))


def kernel(x, edge_index, batch, W_conv1, b_conv1, W_intra, b_intra, W_q, b_q,
           W_att, b_att, Ws1, bs1, Ws2, bs2, Ws3, bs3, Wp1, bp1, Wp2, bp2):
    row0, col0 = edge_index[0], edge_index[1]
    keep = row0 != col0
    drop = jnp.int32(_N)
    row0 = jnp.where(keep, row0, drop)
    col0 = jnp.where(keep, col0, drop)
    loops = jnp.arange(_N, dtype=jnp.int32)
    npad = _EPAD - _E - _N
    padi = jnp.full((npad,), _N, jnp.int32)
    row = jnp.concatenate([row0, loops, padi])
    col = jnp.concatenate([col0, loops, padi])
    ew = jnp.concatenate([jnp.ones((_E + _N,), jnp.float32),
                          jnp.zeros((npad,), jnp.float32)])

    deg = jax.ops.segment_sum(ew, col, num_segments=_N)
    dis = jnp.where(deg > 0, 1.0 / jnp.sqrt(jnp.maximum(deg, 1e-12)), 0.0)
    dis_col = dis.reshape(_N, 1)
    p = lambda v: v.reshape(_PACK_ROWS, 128)

    h0 = _rowscale(_linear(x, W_conv1, jnp.zeros_like(b_conv1)), dis_col)
    h = _scale_bias(jax.ops.segment_sum(h0[row], col, num_segments=_N),
                    dis_col, b_conv1, act="relu")

    l2 = _rowscale(_linear(h, W_intra, jnp.zeros_like(b_intra)), dis_col)
    x_pool = _scale_bias(jax.ops.segment_sum(l2[row], col, num_segments=_N),
                         dis_col, b_intra)

    x_pool_j = x_pool[col]
    x_j = h[col]

    X_q = jax.ops.segment_max(x_pool_j, row, num_segments=_N)
    Mq = _linear(X_q, W_q, b_q)
    Wq12 = jnp.zeros((_H, 128), jnp.float32)
    Wq12 = Wq12.at[:, 0].set(W_att[:_H, 0])
    Wq12 = Wq12.at[:, 1].set(W_att[_H:, 0])
    q1 = _linear(Mq, Wq12, jnp.zeros((128,), jnp.float32))[:, 0]
    q2 = _linear(x_pool, Wq12, jnp.zeros((128,), jnp.float32))[:, 1]
    s2 = pl.pallas_call(
        _leaky_add_k,
        grid=(_PACK_ROWS // _PACK_BLK,),
        in_specs=[
            pl.BlockSpec((_PACK_BLK, 128), lambda i: (i, 0)),
            pl.BlockSpec((_PACK_BLK, 128), lambda i: (i, 0)),
            pl.BlockSpec((1, 1), lambda i: (0, 0)),
        ],
        out_specs=pl.BlockSpec((_PACK_BLK, 128), lambda i: (i, 0)),
        out_shape=jax.ShapeDtypeStruct((_PACK_ROWS, 128), jnp.float32),
    )(p(q1[row]), p(q2[col]), b_att.reshape(1, 1))
    s = s2.reshape(_EPAD)

    m = jax.ops.segment_max(s, row, num_segments=_N)
    e2 = _packed2(_expsub_k, p(s), p(m[row]))
    e = e2.reshape(_EPAD)
    d = jax.ops.segment_sum(e, row, num_segments=_N)
    msg = _attmsg(x_j, e.reshape(_EPAD, 1), d[row].reshape(_EPAD, 1))
    out = jax.ops.segment_sum(msg, row, num_segments=_N)

    Wabc = jnp.zeros((_H, 128), jnp.float32)
    Wabc = Wabc.at[:, 0].set(Ws1[:, 0]).at[:, 1].set(Ws2[:, 0]).at[:, 2].set(Ws3[:, 0])
    babc = jnp.zeros((128,), jnp.float32)
    babc = babc.at[0].set(bs1[0]).at[1].set(bs2[0]).at[2].set(bs3[0])
    abc = _linear(out, Wabc, babc)
    a_v, b_v, c_v = abc[:, 0], abc[:, 1], abc[:, 2]
    lem2 = _packed3(_lemsg_k, p(a_v[row]), p(b_v[col]), p(ew))
    agg = jax.ops.segment_sum(lem2.reshape(_EPAD), row, num_segments=_N)
    fitness = jax.nn.sigmoid(agg + c_v)

    k = int(math.ceil(_RATIO * _N))
    _, perm = jax.lax.top_k(fitness, k)
    xp = out[perm] * fitness[perm][:, None]
    batch_p = batch[perm]
    nb = 1
    cnt = jax.ops.segment_sum(jnp.ones((k,), jnp.float32), batch_p, num_segments=nb)
    x_mean = jax.ops.segment_sum(xp, batch_p, num_segments=nb) / jnp.maximum(cnt, 1.0)[:, None]
    x_max = jax.ops.segment_max(xp, batch_p, num_segments=nb)
    xs = jnp.concatenate([x_mean, x_max], axis=-1)
    logits = jax.nn.relu(xs @ Wp1 + bp1) @ Wp2 + bp2
    pred = jax.nn.log_softmax(logits, axis=-1)
    return pred[..., None]
